# baseline (speedup 1.0000x reference)
import functools

import jax
import jax.numpy as jnp
from jax import lax
from jax.experimental import pallas as pl
from jax.experimental.pallas import tpu as pltpu
from jax.experimental.pallas import tpu_sc as plsc

VOCAB = 1000000
D = 64
DF = 128
B_TOK = 4096 * 200
NC, NS, L = 2, 16, 16
NW = NC * NS
PER_W = B_TOK // NW
CHUNK = 80
NCHUNK = PER_W // CHUNK
NBUF = 4
NGROUP = NCHUNK // NBUF


def _embed_kernel(fat_hbm, idx_hbm, out_hbm, idx_v, *bufs):
    rows = bufs[:NBUF]
    comp = bufs[NBUF:2 * NBUF]
    gsem = bufs[2 * NBUF:3 * NBUF]
    osem = bufs[3 * NBUF:4 * NBUF]

    wid = lax.axis_index("s") * NC + lax.axis_index("c")
    base = wid * PER_W

    pltpu.sync_copy(idx_hbm.at[pl.ds(base, PER_W)], idx_v)

    def gather_start(b, j):
        src = fat_hbm.at[idx_v.at[pl.ds(j * CHUNK, CHUNK)]]
        pltpu.make_async_copy(src, rows[b], gsem[b]).start()

    for b in range(NBUF):
        gather_start(b, b)

    def group(g, _):
        for b in range(NBUF):
            j = g * NBUF + b
            src = fat_hbm.at[idx_v.at[pl.ds(j * CHUNK, CHUNK)]]
            pltpu.make_async_copy(src, rows[b], gsem[b]).wait()

            @plsc.parallel_loop(0, CHUNK, step=4)
            def scale_row(i):
                for rr in range(4):
                    for k in range(D // L):
                        sl = pl.ds(k * L, L)
                        comp[b][i + rr, sl] = rows[b][i + rr, sl] * 8.0

            dst = out_hbm.at[pl.ds(base + j * CHUNK, CHUNK)]
            pltpu.make_async_copy(comp[b], dst, osem[b]).start()

        for b in range(NBUF):
            j = g * NBUF + b
            dst = out_hbm.at[pl.ds(base + j * CHUNK, CHUNK)]
            pltpu.make_async_copy(comp[b], dst, osem[b]).wait()

            @pl.when(g + 1 < NGROUP)
            def _():
                gather_start(b, (g + 1) * NBUF + b)

        return 0

    lax.fori_loop(0, NGROUP, group, 0)


@jax.jit
def _embed(fat, idx):
    mesh = plsc.VectorSubcoreMesh(core_axis_name="c", subcore_axis_name="s")
    f = functools.partial(
        pl.kernel,
        out_type=jax.ShapeDtypeStruct((B_TOK, D), jnp.float32),
        mesh=mesh,
        scratch_types=(
            [pltpu.VMEM((PER_W,), jnp.int32)]
            + [pltpu.VMEM((CHUNK, DF), jnp.float32) for _ in range(NBUF)]
            + [pltpu.VMEM((CHUNK, D), jnp.float32) for _ in range(NBUF)]
            + [pltpu.SemaphoreType.DMA for _ in range(2 * NBUF)]
        ),
        compiler_params=pltpu.CompilerParams(use_tc_tiling_on_sc=True),
    )(_embed_kernel)
    return f(fat, idx)


def kernel(x, table):
    fat = jnp.pad(table, ((0, 0), (0, DF - D)))
    idx = x.reshape(B_TOK).astype(jnp.int32)
    out = _embed(fat, idx)
    return out.reshape(x.shape[0], x.shape[1], D)
